# baseline (speedup 1.0000x reference)
import functools

import jax
import jax.numpy as jnp
from jax import lax
from jax.experimental import pallas as pl
from jax.experimental.pallas import tpu as pltpu
from jax.experimental.pallas import tpu_sc as plsc

N_NODES = 10000
N_EDGES = 160000
D_IN = 256
D_OUT = 256
N_ETYPES = 4

NC = 2
NS = 16
LANES = 16

CHUNK = 128
NROWS = 1280
E_PAD = NROWS * CHUNK
ROWS_PER_TILE = NROWS // NS
NSEG = 2
SEG_ROWS = ROWS_PER_TILE // NSEG
NBUF = 2
ACC_ROWS = 10240
STRIPE = ACC_ROWS // NS
HALF = D_OUT // 2

BN = 1000
EBLK = 320


def _tc_table_body(x_ref, w_ref, b_ref, o_ref):
    acc = lax.dot_general(
        x_ref[...], w_ref[0],
        dimension_numbers=(((1,), (1,)), ((), ())),
        preferred_element_type=jnp.float32,
    )
    o_ref[0] = acc + b_ref[0, 0, 0][None, :]


def _make_table(x, Ws, bs):
    bs4 = jnp.broadcast_to(
        bs.reshape(N_ETYPES, NC, HALF).transpose(1, 0, 2)[:, :, None, :],
        (NC, N_ETYPES, 8, HALF))
    nb = N_NODES // BN
    grid = (nb, NC, N_ETYPES)
    H2 = pl.pallas_call(
        _tc_table_body,
        grid=grid,
        in_specs=[
            pl.BlockSpec((BN, D_IN), lambda n, c, t: (n, 0)),
            pl.BlockSpec((1, HALF, D_IN), lambda n, c, t: (t, c, 0)),
            pl.BlockSpec((1, 1, 8, HALF), lambda n, c, t: (c, t, 0, 0)),
        ],
        out_specs=pl.BlockSpec((1, BN, HALF), lambda n, c, t: (c, t * nb + n, 0)),
        out_shape=jax.ShapeDtypeStruct((NC, N_ETYPES * N_NODES, HALF), jnp.float32),
    )(x, Ws, bs4)
    return H2.reshape(NC * N_ETYPES * N_NODES, HALF)


def _tc_gidx_body(src_ref, et_ref, o_ref):
    c = pl.program_id(0)
    o_ref[0] = et_ref[...] * N_NODES + src_ref[...] + c * (N_ETYPES * N_NODES)


def _make_gidx(src2, et2):
    nblk = NROWS // EBLK
    gidx = pl.pallas_call(
        _tc_gidx_body,
        grid=(NC, nblk),
        in_specs=[
            pl.BlockSpec((EBLK, CHUNK), lambda c, j: (j, 0)),
            pl.BlockSpec((EBLK, CHUNK), lambda c, j: (j, 0)),
        ],
        out_specs=pl.BlockSpec((1, EBLK, CHUNK), lambda c, j: (c, j, 0)),
        out_shape=jax.ShapeDtypeStruct((NC, NROWS, CHUNK), jnp.int32),
    )(src2, et2)
    return gidx.reshape(NC * NROWS, CHUNK)


def _sc_kernel_body(h_hbm, gidx_hbm, dst_hbm, out_hbm,
                    gidx_v, dst_v, rows_v, acc, gsem0, gsem1):
    c = lax.axis_index("c")
    s = lax.axis_index("s")
    base = s * ROWS_PER_TILE
    gsems = [gsem0, gsem1]

    def _zb(i, _):
        rows_v[0, i // 8, pl.ds((i % 8) * LANES, LANES)] = jnp.zeros((LANES,), jnp.float32)
        return 0
    lax.fori_loop(0, CHUNK * 8, _zb, 0)

    def _za(m, _):
        pltpu.sync_copy(rows_v.at[0], acc.at[pl.ds(s * STRIPE + m * CHUNK, CHUNK)])
        return 0
    lax.fori_loop(0, STRIPE // CHUNK, _za, 0)

    plsc.subcore_barrier()

    def _gather(ch, b):
        pltpu.async_copy(h_hbm.at[gidx_v.at[ch]], rows_v.at[b], gsems[b])

    def _gather_wait(ch, b):
        pltpu.make_async_copy(h_hbm.at[gidx_v.at[ch]], rows_v.at[b], gsems[b]).wait()

    for seg in range(NSEG):
        seg_base = base + seg * SEG_ROWS
        pltpu.sync_copy(gidx_hbm.at[pl.ds(c * NROWS + seg_base, SEG_ROWS)], gidx_v)
        pltpu.sync_copy(dst_hbm.at[pl.ds(seg_base, SEG_ROWS)], dst_v)

        for b in range(NBUF):
            _gather(b, b)

        def _mb(j, _):
            for b in range(NBUF):
                ch = j * NBUF + b
                _gather_wait(ch, b)
                pltpu.sync_copy(rows_v.at[b], acc.at[dst_v.at[ch]], add=True)

                @pl.when(ch + NBUF < SEG_ROWS)
                def _():
                    _gather(ch + NBUF, b)
            return 0
        lax.fori_loop(0, SEG_ROWS // NBUF, _mb, 0)

    plsc.subcore_barrier()
    r0 = s * STRIPE

    @pl.when(s < NS - 1)
    def _():
        pltpu.sync_copy(acc.at[pl.ds(r0, STRIPE)],
                        out_hbm.at[pl.ds(r0, STRIPE), pl.ds(c * HALF, HALF)])

    @pl.when(s == NS - 1)
    def _():
        pltpu.sync_copy(acc.at[pl.ds(r0, N_NODES - (NS - 1) * STRIPE)],
                        out_hbm.at[pl.ds(r0, N_NODES - (NS - 1) * STRIPE),
                                   pl.ds(c * HALF, HALF)])


_sc_kernel = functools.partial(
    pl.kernel,
    out_type=jax.ShapeDtypeStruct((N_NODES, D_OUT), jnp.float32),
    mesh=plsc.VectorSubcoreMesh(core_axis_name="c", subcore_axis_name="s",
                                num_cores=NC, num_subcores=NS),
    scratch_types=[
        pltpu.VMEM((SEG_ROWS, CHUNK), jnp.int32),
        pltpu.VMEM((SEG_ROWS, CHUNK), jnp.int32),
        pltpu.VMEM((NBUF, CHUNK, HALF), jnp.float32),
        pltpu.VMEM_SHARED((ACC_ROWS, HALF), jnp.float32),
        pltpu.SemaphoreType.DMA, pltpu.SemaphoreType.DMA,
    ],
)(_sc_kernel_body)


@jax.jit
def kernel(x, edge_index, edge_types, Ws, bs):
    src = edge_index[0].astype(jnp.int32)
    dst = edge_index[1].astype(jnp.int32)
    et = edge_types.astype(jnp.int32)

    pad = E_PAD - N_EDGES
    src2 = jnp.concatenate([src, jnp.zeros((pad,), jnp.int32)]).reshape(-1, CHUNK)
    et2 = jnp.concatenate([et, jnp.zeros((pad,), jnp.int32)]).reshape(-1, CHUNK)
    dst_p = jnp.concatenate([dst, jnp.full((pad,), N_NODES, jnp.int32)]).reshape(-1, CHUNK)

    table = _make_table(x, Ws, bs)
    gidx = _make_gidx(src2, et2)
    return _sc_kernel(table, gidx, dst_p)


# speedup vs baseline: 1.0094x; 1.0094x over previous
import functools

import jax
import jax.numpy as jnp
from jax import lax
from jax.experimental import pallas as pl
from jax.experimental.pallas import tpu as pltpu
from jax.experimental.pallas import tpu_sc as plsc

N_NODES = 10000
N_EDGES = 160000
D_IN = 256
D_OUT = 256
N_ETYPES = 4

NC = 2
NS = 16
LANES = 16

CHUNK = 128
NROWS = 1280
E_PAD = NROWS * CHUNK
ROWS_PER_TILE = NROWS // NS
NSEG = 2
SEG_ROWS = ROWS_PER_TILE // NSEG
NBUF = 2
ACC_ROWS = 10240
STRIPE = ACC_ROWS // NS
HALF = D_OUT // 2

BN = 1000
EBLK = 320


def _tc_body(x_ref, w_ref, b_ref, src_ref, et_ref, h_ref, g_ref):
    acc = lax.dot_general(
        x_ref[...], w_ref[0],
        dimension_numbers=(((1,), (1,)), ((), ())),
        preferred_element_type=jnp.float32,
    )
    h_ref[0] = acc + b_ref[0, 0, 0][None, :]
    c = pl.program_id(1)
    g_ref[0] = et_ref[...] * N_NODES + src_ref[...] + c * (N_ETYPES * N_NODES)


def _make_table_gidx(x, Ws, bs, src2, et2):
    bs4 = jnp.broadcast_to(
        bs.reshape(N_ETYPES, NC, HALF).transpose(1, 0, 2)[:, :, None, :],
        (NC, N_ETYPES, 8, HALF))
    nb = N_NODES // BN
    erows = NROWS // nb
    grid = (nb, NC, N_ETYPES)
    H2, gidx = pl.pallas_call(
        _tc_body,
        grid=grid,
        in_specs=[
            pl.BlockSpec((BN, D_IN), lambda n, c, t: (n, 0)),
            pl.BlockSpec((1, HALF, D_IN), lambda n, c, t: (t, c, 0)),
            pl.BlockSpec((1, 1, 8, HALF), lambda n, c, t: (c, t, 0, 0)),
            pl.BlockSpec((erows, CHUNK), lambda n, c, t: (n, 0)),
            pl.BlockSpec((erows, CHUNK), lambda n, c, t: (n, 0)),
        ],
        out_specs=[
            pl.BlockSpec((1, BN, HALF), lambda n, c, t: (c, t * nb + n, 0)),
            pl.BlockSpec((1, erows, CHUNK), lambda n, c, t: (c, n, 0)),
        ],
        out_shape=[
            jax.ShapeDtypeStruct((NC, N_ETYPES * N_NODES, HALF), jnp.float32),
            jax.ShapeDtypeStruct((NC, NROWS, CHUNK), jnp.int32),
        ],
    )(x, Ws, bs4, src2, et2)
    return (H2.reshape(NC * N_ETYPES * N_NODES, HALF),
            gidx.reshape(NC * NROWS, CHUNK))


def _sc_kernel_body(h_hbm, gidx_hbm, dst_hbm, out_hbm,
                    gidx_v, dst_v, rows_v, acc, gsem0, gsem1):
    c = lax.axis_index("c")
    s = lax.axis_index("s")
    base = s * ROWS_PER_TILE
    gsems = [gsem0, gsem1]

    def _zb(i, _):
        rows_v[0, i // 8, pl.ds((i % 8) * LANES, LANES)] = jnp.zeros((LANES,), jnp.float32)
        return 0
    lax.fori_loop(0, CHUNK * 8, _zb, 0)

    def _za(m, _):
        pltpu.sync_copy(rows_v.at[0], acc.at[pl.ds(s * STRIPE + m * CHUNK, CHUNK)])
        return 0
    lax.fori_loop(0, STRIPE // CHUNK, _za, 0)

    plsc.subcore_barrier()

    def _gather(ch, b):
        pltpu.async_copy(h_hbm.at[gidx_v.at[ch]], rows_v.at[b], gsems[b])

    def _gather_wait(ch, b):
        pltpu.make_async_copy(h_hbm.at[gidx_v.at[ch]], rows_v.at[b], gsems[b]).wait()

    for seg in range(NSEG):
        seg_base = base + seg * SEG_ROWS
        pltpu.sync_copy(gidx_hbm.at[pl.ds(c * NROWS + seg_base, SEG_ROWS)], gidx_v)
        pltpu.sync_copy(dst_hbm.at[pl.ds(seg_base, SEG_ROWS)], dst_v)

        for b in range(NBUF):
            _gather(b, b)

        def _mb(j, _):
            for b in range(NBUF):
                ch = j * NBUF + b
                _gather_wait(ch, b)
                pltpu.sync_copy(rows_v.at[b], acc.at[dst_v.at[ch]], add=True)

                @pl.when(ch + NBUF < SEG_ROWS)
                def _():
                    _gather(ch + NBUF, b)
            return 0
        lax.fori_loop(0, SEG_ROWS // NBUF, _mb, 0)

    plsc.subcore_barrier()
    pltpu.sync_copy(acc.at[pl.ds(s * STRIPE, STRIPE)],
                    out_hbm.at[pl.ds(c * ACC_ROWS + s * STRIPE, STRIPE)])


_sc_kernel = functools.partial(
    pl.kernel,
    out_type=jax.ShapeDtypeStruct((NC * ACC_ROWS, HALF), jnp.float32),
    mesh=plsc.VectorSubcoreMesh(core_axis_name="c", subcore_axis_name="s",
                                num_cores=NC, num_subcores=NS),
    scratch_types=[
        pltpu.VMEM((SEG_ROWS, CHUNK), jnp.int32),
        pltpu.VMEM((SEG_ROWS, CHUNK), jnp.int32),
        pltpu.VMEM((NBUF, CHUNK, HALF), jnp.float32),
        pltpu.VMEM_SHARED((ACC_ROWS, HALF), jnp.float32),
        pltpu.SemaphoreType.DMA, pltpu.SemaphoreType.DMA,
    ],
)(_sc_kernel_body)


@jax.jit
def kernel(x, edge_index, edge_types, Ws, bs):
    src = edge_index[0].astype(jnp.int32)
    dst = edge_index[1].astype(jnp.int32)
    et = edge_types.astype(jnp.int32)

    pad = E_PAD - N_EDGES
    src2 = jnp.concatenate([src, jnp.zeros((pad,), jnp.int32)]).reshape(-1, CHUNK)
    et2 = jnp.concatenate([et, jnp.zeros((pad,), jnp.int32)]).reshape(-1, CHUNK)
    dst_p = jnp.concatenate([dst, jnp.full((pad,), N_NODES, jnp.int32)]).reshape(-1, CHUNK)

    table, gidx = _make_table_gidx(x, Ws, bs, src2, et2)
    o = _sc_kernel(table, gidx, dst_p)
    return jnp.concatenate([o[:N_NODES], o[ACC_ROWS:ACC_ROWS + N_NODES]], axis=1)


# speedup vs baseline: 1.0683x; 1.0583x over previous
import functools

import jax
import jax.numpy as jnp
from jax import lax
from jax.experimental import pallas as pl
from jax.experimental.pallas import tpu as pltpu
from jax.experimental.pallas import tpu_sc as plsc

N_NODES = 10000
N_EDGES = 160000
D_IN = 256
D_OUT = 256
N_ETYPES = 4

NC = 2
NS = 16
LANES = 16

CHUNK = 128
NROWS = 1280
E_PAD = NROWS * CHUNK
ROWS_PER_TILE = NROWS // NS
NSEG = 2
SEG_ROWS = ROWS_PER_TILE // NSEG
NBUF = 2
ACC_ROWS = 10240
STRIPE = ACC_ROWS // NS
HALF = D_OUT // 2

BN = 2000
EBLK = 320


def _tc_body(x_ref, w_ref, b_ref, src_ref, et_ref, h_ref, g_ref):
    acc = lax.dot_general(
        x_ref[...], w_ref[0],
        dimension_numbers=(((1,), (1,)), ((), ())),
        preferred_element_type=jnp.float32,
    )
    h_ref[0] = acc + b_ref[0, 0, 0][None, :]
    c = pl.program_id(1)
    g_ref[0] = et_ref[...] * N_NODES + src_ref[...] + c * (N_ETYPES * N_NODES)


def _make_table_gidx(x, Ws, bs, src2, et2):
    bs4 = jnp.broadcast_to(
        bs.reshape(N_ETYPES, NC, HALF).transpose(1, 0, 2)[:, :, None, :],
        (NC, N_ETYPES, 8, HALF))
    nb = N_NODES // BN
    erows = NROWS // nb
    grid = (nb, NC, N_ETYPES)
    H2, gidx = pl.pallas_call(
        _tc_body,
        grid=grid,
        in_specs=[
            pl.BlockSpec((BN, D_IN), lambda n, c, t: (n, 0)),
            pl.BlockSpec((1, HALF, D_IN), lambda n, c, t: (t, c, 0)),
            pl.BlockSpec((1, 1, 8, HALF), lambda n, c, t: (c, t, 0, 0)),
            pl.BlockSpec((erows, CHUNK), lambda n, c, t: (n, 0)),
            pl.BlockSpec((erows, CHUNK), lambda n, c, t: (n, 0)),
        ],
        out_specs=[
            pl.BlockSpec((1, BN, HALF), lambda n, c, t: (c, t * nb + n, 0)),
            pl.BlockSpec((1, erows, CHUNK), lambda n, c, t: (c, n, 0)),
        ],
        out_shape=[
            jax.ShapeDtypeStruct((NC, N_ETYPES * N_NODES, HALF), jnp.float32),
            jax.ShapeDtypeStruct((NC, NROWS, CHUNK), jnp.int32),
        ],
    )(x, Ws, bs4, src2, et2)
    return (H2.reshape(NC * N_ETYPES * N_NODES, HALF),
            gidx.reshape(NC * NROWS, CHUNK))


def _sc_kernel_body(h_hbm, gidx_hbm, dst_hbm, out_hbm,
                    gidx_v, dst_v, rows_v, acc, gsem0, gsem1):
    c = lax.axis_index("c")
    s = lax.axis_index("s")
    base = s * ROWS_PER_TILE
    gsems = [gsem0, gsem1]

    def _zb(i, _):
        rows_v[0, i // 8, pl.ds((i % 8) * LANES, LANES)] = jnp.zeros((LANES,), jnp.float32)
        return 0
    lax.fori_loop(0, CHUNK * 8, _zb, 0)

    def _za(m, _):
        pltpu.sync_copy(rows_v.at[0], acc.at[pl.ds(s * STRIPE + m * CHUNK, CHUNK)])
        return 0
    lax.fori_loop(0, STRIPE // CHUNK, _za, 0)

    plsc.subcore_barrier()

    def _gather(ch, b):
        pltpu.async_copy(h_hbm.at[gidx_v.at[ch]], rows_v.at[b], gsems[b])

    def _gather_wait(ch, b):
        pltpu.make_async_copy(h_hbm.at[gidx_v.at[ch]], rows_v.at[b], gsems[b]).wait()

    for seg in range(NSEG):
        seg_base = base + seg * SEG_ROWS
        pltpu.sync_copy(gidx_hbm.at[pl.ds(c * NROWS + seg_base, SEG_ROWS)], gidx_v)
        pltpu.sync_copy(dst_hbm.at[pl.ds(seg_base, SEG_ROWS)], dst_v)

        for b in range(NBUF):
            _gather(b, b)

        def _mb(j, _):
            for b in range(NBUF):
                ch = j * NBUF + b
                _gather_wait(ch, b)
                pltpu.sync_copy(rows_v.at[b], acc.at[dst_v.at[ch]], add=True)

                @pl.when(ch + NBUF < SEG_ROWS)
                def _():
                    _gather(ch + NBUF, b)
            return 0
        lax.fori_loop(0, SEG_ROWS // NBUF, _mb, 0)

    plsc.subcore_barrier()
    pltpu.sync_copy(acc.at[pl.ds(s * STRIPE, STRIPE)],
                    out_hbm.at[pl.ds(c * ACC_ROWS + s * STRIPE, STRIPE)])


_sc_kernel = functools.partial(
    pl.kernel,
    out_type=jax.ShapeDtypeStruct((NC * ACC_ROWS, HALF), jnp.float32),
    mesh=plsc.VectorSubcoreMesh(core_axis_name="c", subcore_axis_name="s",
                                num_cores=NC, num_subcores=NS),
    scratch_types=[
        pltpu.VMEM((SEG_ROWS, CHUNK), jnp.int32),
        pltpu.VMEM((SEG_ROWS, CHUNK), jnp.int32),
        pltpu.VMEM((NBUF, CHUNK, HALF), jnp.float32),
        pltpu.VMEM_SHARED((ACC_ROWS, HALF), jnp.float32),
        pltpu.SemaphoreType.DMA, pltpu.SemaphoreType.DMA,
    ],
)(_sc_kernel_body)


@jax.jit
def kernel(x, edge_index, edge_types, Ws, bs):
    src = edge_index[0].astype(jnp.int32)
    dst = edge_index[1].astype(jnp.int32)
    et = edge_types.astype(jnp.int32)

    pad = E_PAD - N_EDGES
    src2 = jnp.concatenate([src, jnp.zeros((pad,), jnp.int32)]).reshape(-1, CHUNK)
    et2 = jnp.concatenate([et, jnp.zeros((pad,), jnp.int32)]).reshape(-1, CHUNK)
    dst_p = jnp.concatenate([dst, jnp.full((pad,), N_NODES, jnp.int32)]).reshape(-1, CHUNK)

    table, gidx = _make_table_gidx(x, Ws, bs, src2, et2)
    o = _sc_kernel(table, gidx, dst_p)
    return jnp.concatenate([o[:N_NODES], o[ACC_ROWS:ACC_ROWS + N_NODES]], axis=1)


# speedup vs baseline: 1.1096x; 1.0387x over previous
import functools

import jax
import jax.numpy as jnp
from jax import lax
from jax.experimental import pallas as pl
from jax.experimental.pallas import tpu as pltpu
from jax.experimental.pallas import tpu_sc as plsc

N_NODES = 10000
N_EDGES = 160000
D_IN = 256
D_OUT = 256
N_ETYPES = 4

NC = 2
NS = 16
LANES = 16

CHUNK = 128
NROWS = 1280
E_PAD = NROWS * CHUNK
ROWS_PER_TILE = NROWS // NS
NSEG = 2
SEG_ROWS = ROWS_PER_TILE // NSEG
NBUF = 2
ACC_ROWS = 10240
STRIPE = ACC_ROWS // NS
HALF = D_OUT // 2

BN = 5000
EBLK = 320


def _tc_body(x_ref, w_ref, b_ref, src_ref, et_ref, h_ref, g_ref):
    acc = lax.dot_general(
        x_ref[...], w_ref[0],
        dimension_numbers=(((1,), (1,)), ((), ())),
        preferred_element_type=jnp.float32,
    )
    h_ref[0] = acc + b_ref[0, 0, 0][None, :]
    c = pl.program_id(1)
    g_ref[0] = et_ref[...] * N_NODES + src_ref[...] + c * (N_ETYPES * N_NODES)


def _make_table_gidx(x, Ws, bs, src2, et2):
    bs4 = jnp.broadcast_to(
        bs.reshape(N_ETYPES, NC, HALF).transpose(1, 0, 2)[:, :, None, :],
        (NC, N_ETYPES, 8, HALF))
    nb = N_NODES // BN
    erows = NROWS // nb
    grid = (nb, NC, N_ETYPES)
    H2, gidx = pl.pallas_call(
        _tc_body,
        grid=grid,
        in_specs=[
            pl.BlockSpec((BN, D_IN), lambda n, c, t: (n, 0)),
            pl.BlockSpec((1, HALF, D_IN), lambda n, c, t: (t, c, 0)),
            pl.BlockSpec((1, 1, 8, HALF), lambda n, c, t: (c, t, 0, 0)),
            pl.BlockSpec((erows, CHUNK), lambda n, c, t: (n, 0)),
            pl.BlockSpec((erows, CHUNK), lambda n, c, t: (n, 0)),
        ],
        out_specs=[
            pl.BlockSpec((1, BN, HALF), lambda n, c, t: (c, t * nb + n, 0)),
            pl.BlockSpec((1, erows, CHUNK), lambda n, c, t: (c, n, 0)),
        ],
        out_shape=[
            jax.ShapeDtypeStruct((NC, N_ETYPES * N_NODES, HALF), jnp.float32),
            jax.ShapeDtypeStruct((NC, NROWS, CHUNK), jnp.int32),
        ],
    )(x, Ws, bs4, src2, et2)
    return (H2.reshape(NC * N_ETYPES * N_NODES, HALF),
            gidx.reshape(NC * NROWS, CHUNK))


def _sc_kernel_body(h_hbm, gidx_hbm, dst_hbm, out_hbm,
                    gidx_v, dst_v, rows_v, acc, gsem0, gsem1):
    c = lax.axis_index("c")
    s = lax.axis_index("s")
    base = s * ROWS_PER_TILE
    gsems = [gsem0, gsem1]

    def _zb(i, _):
        rows_v[0, i // 8, pl.ds((i % 8) * LANES, LANES)] = jnp.zeros((LANES,), jnp.float32)
        return 0
    lax.fori_loop(0, CHUNK * 8, _zb, 0)

    def _za(m, _):
        pltpu.sync_copy(rows_v.at[0], acc.at[pl.ds(s * STRIPE + m * CHUNK, CHUNK)])
        return 0
    lax.fori_loop(0, STRIPE // CHUNK, _za, 0)

    plsc.subcore_barrier()

    def _gather(ch, b):
        pltpu.async_copy(h_hbm.at[gidx_v.at[ch]], rows_v.at[b], gsems[b])

    def _gather_wait(ch, b):
        pltpu.make_async_copy(h_hbm.at[gidx_v.at[ch]], rows_v.at[b], gsems[b]).wait()

    for seg in range(NSEG):
        seg_base = base + seg * SEG_ROWS
        pltpu.sync_copy(gidx_hbm.at[pl.ds(c * NROWS + seg_base, SEG_ROWS)], gidx_v)
        pltpu.sync_copy(dst_hbm.at[pl.ds(seg_base, SEG_ROWS)], dst_v)

        for b in range(NBUF):
            _gather(b, b)

        def _mb(j, _):
            for b in range(NBUF):
                ch = j * NBUF + b
                _gather_wait(ch, b)
                pltpu.sync_copy(rows_v.at[b], acc.at[dst_v.at[ch]], add=True)

                @pl.when(ch + NBUF < SEG_ROWS)
                def _():
                    _gather(ch + NBUF, b)
            return 0
        lax.fori_loop(0, SEG_ROWS // NBUF, _mb, 0)

    plsc.subcore_barrier()
    pltpu.sync_copy(acc.at[pl.ds(s * STRIPE, STRIPE)],
                    out_hbm.at[pl.ds(c * ACC_ROWS + s * STRIPE, STRIPE)])


_sc_kernel = functools.partial(
    pl.kernel,
    out_type=jax.ShapeDtypeStruct((NC * ACC_ROWS, HALF), jnp.float32),
    mesh=plsc.VectorSubcoreMesh(core_axis_name="c", subcore_axis_name="s",
                                num_cores=NC, num_subcores=NS),
    scratch_types=[
        pltpu.VMEM((SEG_ROWS, CHUNK), jnp.int32),
        pltpu.VMEM((SEG_ROWS, CHUNK), jnp.int32),
        pltpu.VMEM((NBUF, CHUNK, HALF), jnp.float32),
        pltpu.VMEM_SHARED((ACC_ROWS, HALF), jnp.float32),
        pltpu.SemaphoreType.DMA, pltpu.SemaphoreType.DMA,
    ],
)(_sc_kernel_body)


@jax.jit
def kernel(x, edge_index, edge_types, Ws, bs):
    src = edge_index[0].astype(jnp.int32)
    dst = edge_index[1].astype(jnp.int32)
    et = edge_types.astype(jnp.int32)

    pad = E_PAD - N_EDGES
    src2 = jnp.concatenate([src, jnp.zeros((pad,), jnp.int32)]).reshape(-1, CHUNK)
    et2 = jnp.concatenate([et, jnp.zeros((pad,), jnp.int32)]).reshape(-1, CHUNK)
    dst_p = jnp.concatenate([dst, jnp.full((pad,), N_NODES, jnp.int32)]).reshape(-1, CHUNK)

    table, gidx = _make_table_gidx(x, Ws, bs, src2, et2)
    o = _sc_kernel(table, gidx, dst_p)
    return jnp.concatenate([o[:N_NODES], o[ACC_ROWS:ACC_ROWS + N_NODES]], axis=1)


# speedup vs baseline: 1.1251x; 1.0139x over previous
import functools

import jax
import jax.numpy as jnp
from jax import lax
from jax.experimental import pallas as pl
from jax.experimental.pallas import tpu as pltpu
from jax.experimental.pallas import tpu_sc as plsc

N_NODES = 10000
N_EDGES = 160000
D_IN = 256
D_OUT = 256
N_ETYPES = 4

NC = 2
NS = 16
LANES = 16

CHUNK = 128
NROWS = 1280
E_PAD = NROWS * CHUNK
ROWS_PER_TILE = NROWS // NS
NSEG = 2
SEG_ROWS = ROWS_PER_TILE // NSEG
NBUF = 2
ACC_ROWS = 10240
STRIPE = ACC_ROWS // NS
HALF = D_OUT // 2

BN = 10000
EBLK = 320


def _tc_body(x_ref, w_ref, b_ref, src_ref, et_ref, h_ref, g_ref):
    acc = lax.dot_general(
        x_ref[...], w_ref[0],
        dimension_numbers=(((1,), (1,)), ((), ())),
        preferred_element_type=jnp.float32,
    )
    h_ref[0] = acc + b_ref[0, 0, 0][None, :]
    c = pl.program_id(1)
    g_ref[0] = et_ref[...] * N_NODES + src_ref[...] + c * (N_ETYPES * N_NODES)


def _make_table_gidx(x, Ws, bs, src2, et2):
    bs4 = jnp.broadcast_to(
        bs.reshape(N_ETYPES, NC, HALF).transpose(1, 0, 2)[:, :, None, :],
        (NC, N_ETYPES, 8, HALF))
    nb = N_NODES // BN
    erows = NROWS // nb
    grid = (nb, NC, N_ETYPES)
    H2, gidx = pl.pallas_call(
        _tc_body,
        grid=grid,
        in_specs=[
            pl.BlockSpec((BN, D_IN), lambda n, c, t: (n, 0)),
            pl.BlockSpec((1, HALF, D_IN), lambda n, c, t: (t, c, 0)),
            pl.BlockSpec((1, 1, 8, HALF), lambda n, c, t: (c, t, 0, 0)),
            pl.BlockSpec((erows, CHUNK), lambda n, c, t: (n, 0)),
            pl.BlockSpec((erows, CHUNK), lambda n, c, t: (n, 0)),
        ],
        out_specs=[
            pl.BlockSpec((1, BN, HALF), lambda n, c, t: (c, t * nb + n, 0)),
            pl.BlockSpec((1, erows, CHUNK), lambda n, c, t: (c, n, 0)),
        ],
        out_shape=[
            jax.ShapeDtypeStruct((NC, N_ETYPES * N_NODES, HALF), jnp.float32),
            jax.ShapeDtypeStruct((NC, NROWS, CHUNK), jnp.int32),
        ],
    )(x, Ws, bs4, src2, et2)
    return (H2.reshape(NC * N_ETYPES * N_NODES, HALF),
            gidx.reshape(NC * NROWS, CHUNK))


def _sc_kernel_body(h_hbm, gidx_hbm, dst_hbm, out_hbm,
                    gidx_v, dst_v, rows_v, acc, gsem0, gsem1):
    c = lax.axis_index("c")
    s = lax.axis_index("s")
    base = s * ROWS_PER_TILE
    gsems = [gsem0, gsem1]

    def _zb(i, _):
        rows_v[0, i // 8, pl.ds((i % 8) * LANES, LANES)] = jnp.zeros((LANES,), jnp.float32)
        return 0
    lax.fori_loop(0, CHUNK * 8, _zb, 0)

    def _za(m, _):
        pltpu.sync_copy(rows_v.at[0], acc.at[pl.ds(s * STRIPE + m * CHUNK, CHUNK)])
        return 0
    lax.fori_loop(0, STRIPE // CHUNK, _za, 0)

    plsc.subcore_barrier()

    def _gather(ch, b):
        pltpu.async_copy(h_hbm.at[gidx_v.at[ch]], rows_v.at[b], gsems[b])

    def _gather_wait(ch, b):
        pltpu.make_async_copy(h_hbm.at[gidx_v.at[ch]], rows_v.at[b], gsems[b]).wait()

    for seg in range(NSEG):
        seg_base = base + seg * SEG_ROWS
        pltpu.sync_copy(gidx_hbm.at[pl.ds(c * NROWS + seg_base, SEG_ROWS)], gidx_v)
        pltpu.sync_copy(dst_hbm.at[pl.ds(seg_base, SEG_ROWS)], dst_v)

        for b in range(NBUF):
            _gather(b, b)

        def _mb(j, _):
            for b in range(NBUF):
                ch = j * NBUF + b
                _gather_wait(ch, b)
                pltpu.sync_copy(rows_v.at[b], acc.at[dst_v.at[ch]], add=True)

                @pl.when(ch + NBUF < SEG_ROWS)
                def _():
                    _gather(ch + NBUF, b)
            return 0
        lax.fori_loop(0, SEG_ROWS // NBUF, _mb, 0)

    plsc.subcore_barrier()
    pltpu.sync_copy(acc.at[pl.ds(s * STRIPE, STRIPE)],
                    out_hbm.at[pl.ds(c * ACC_ROWS + s * STRIPE, STRIPE)])


_sc_kernel = functools.partial(
    pl.kernel,
    out_type=jax.ShapeDtypeStruct((NC * ACC_ROWS, HALF), jnp.float32),
    mesh=plsc.VectorSubcoreMesh(core_axis_name="c", subcore_axis_name="s",
                                num_cores=NC, num_subcores=NS),
    scratch_types=[
        pltpu.VMEM((SEG_ROWS, CHUNK), jnp.int32),
        pltpu.VMEM((SEG_ROWS, CHUNK), jnp.int32),
        pltpu.VMEM((NBUF, CHUNK, HALF), jnp.float32),
        pltpu.VMEM_SHARED((ACC_ROWS, HALF), jnp.float32),
        pltpu.SemaphoreType.DMA, pltpu.SemaphoreType.DMA,
    ],
)(_sc_kernel_body)


@jax.jit
def kernel(x, edge_index, edge_types, Ws, bs):
    src = edge_index[0].astype(jnp.int32)
    dst = edge_index[1].astype(jnp.int32)
    et = edge_types.astype(jnp.int32)

    pad = E_PAD - N_EDGES
    src2 = jnp.concatenate([src, jnp.zeros((pad,), jnp.int32)]).reshape(-1, CHUNK)
    et2 = jnp.concatenate([et, jnp.zeros((pad,), jnp.int32)]).reshape(-1, CHUNK)
    dst_p = jnp.concatenate([dst, jnp.full((pad,), N_NODES, jnp.int32)]).reshape(-1, CHUNK)

    table, gidx = _make_table_gidx(x, Ws, bs, src2, et2)
    o = _sc_kernel(table, gidx, dst_p)
    return jnp.concatenate([o[:N_NODES], o[ACC_ROWS:ACC_ROWS + N_NODES]], axis=1)


# speedup vs baseline: 1.1275x; 1.0021x over previous
import functools

import jax
import jax.numpy as jnp
from jax import lax
from jax.experimental import pallas as pl
from jax.experimental.pallas import tpu as pltpu
from jax.experimental.pallas import tpu_sc as plsc

N_NODES = 10000
N_EDGES = 160000
D_IN = 256
D_OUT = 256
N_ETYPES = 4

NC = 2
NS = 16
LANES = 16

CHUNK = 128
NROWS = 1280
E_PAD = NROWS * CHUNK
ROWS_PER_TILE = NROWS // NS
NSEG = 2
SEG_ROWS = ROWS_PER_TILE // NSEG
NBUF = 2
ACC_ROWS = 10240
STRIPE = ACC_ROWS // NS
HALF = D_OUT // 2

BN = 10000
EBLK = 320


def _tc_body(x_ref, w_ref, b_ref, src_ref, et_ref, h_ref, g_ref):
    acc = lax.dot_general(
        x_ref[...], w_ref[0],
        dimension_numbers=(((1,), (1,)), ((), ())),
        preferred_element_type=jnp.float32,
    )
    h_ref[0] = acc + b_ref[0, 0, 0][None, :]
    c = pl.program_id(1)
    g_ref[0] = et_ref[...] * N_NODES + src_ref[...] + c * (N_ETYPES * N_NODES)


def _make_table_gidx(x, Ws, bs, src2, et2):
    bs4 = jnp.broadcast_to(
        bs.reshape(N_ETYPES, NC, HALF).transpose(1, 0, 2)[:, :, None, :],
        (NC, N_ETYPES, 8, HALF))
    nb = N_NODES // BN
    erows = NROWS // nb
    grid = (nb, NC, N_ETYPES)
    H2, gidx = pl.pallas_call(
        _tc_body,
        grid=grid,
        in_specs=[
            pl.BlockSpec((BN, D_IN), lambda n, c, t: (n, 0)),
            pl.BlockSpec((1, HALF, D_IN), lambda n, c, t: (t, c, 0)),
            pl.BlockSpec((1, 1, 8, HALF), lambda n, c, t: (c, t, 0, 0)),
            pl.BlockSpec((erows, CHUNK), lambda n, c, t: (n, 0)),
            pl.BlockSpec((erows, CHUNK), lambda n, c, t: (n, 0)),
        ],
        out_specs=[
            pl.BlockSpec((1, BN, HALF), lambda n, c, t: (c, t * nb + n, 0)),
            pl.BlockSpec((1, erows, CHUNK), lambda n, c, t: (c, n, 0)),
        ],
        out_shape=[
            jax.ShapeDtypeStruct((NC, N_ETYPES * N_NODES, HALF), jnp.float32),
            jax.ShapeDtypeStruct((NC, NROWS, CHUNK), jnp.int32),
        ],
    )(x, Ws, bs4, src2, et2)
    return (H2.reshape(NC * N_ETYPES * N_NODES, HALF),
            gidx.reshape(NC * NROWS, CHUNK))


def _sc_kernel_body(h_hbm, gidx_hbm, dst_hbm, out_hbm,
                    gidx_v, dst_v, rows_v, acc, gsem0, gsem1):
    c = lax.axis_index("c")
    s = lax.axis_index("s")
    base = s * ROWS_PER_TILE
    gsems = [gsem0, gsem1]

    def _gather(ch, b):
        pltpu.async_copy(h_hbm.at[gidx_v.at[ch]], rows_v.at[b], gsems[b])

    def _gather_wait(ch, b):
        pltpu.make_async_copy(h_hbm.at[gidx_v.at[ch]], rows_v.at[b], gsems[b]).wait()

    pltpu.sync_copy(gidx_hbm.at[pl.ds(c * NROWS + base, SEG_ROWS)], gidx_v)
    pltpu.sync_copy(dst_hbm.at[pl.ds(base, SEG_ROWS)], dst_v)
    _gather(0, 0)

    def _zb(i, _):
        rows_v[1, i // 8, pl.ds((i % 8) * LANES, LANES)] = jnp.zeros((LANES,), jnp.float32)
        return 0
    lax.fori_loop(0, CHUNK * 8, _zb, 0)

    def _za(m, _):
        pltpu.sync_copy(rows_v.at[1], acc.at[pl.ds(s * STRIPE + m * CHUNK, CHUNK)])
        return 0
    lax.fori_loop(0, STRIPE // CHUNK, _za, 0)

    plsc.subcore_barrier()

    for seg in range(NSEG):
        seg_base = base + seg * SEG_ROWS
        if seg > 0:
            pltpu.sync_copy(gidx_hbm.at[pl.ds(c * NROWS + seg_base, SEG_ROWS)],
                            gidx_v)
            pltpu.sync_copy(dst_hbm.at[pl.ds(seg_base, SEG_ROWS)], dst_v)

        for b in range(NBUF):
            if seg > 0 or b > 0:
                _gather(b, b)

        def _mb(j, _):
            for b in range(NBUF):
                ch = j * NBUF + b
                _gather_wait(ch, b)
                pltpu.sync_copy(rows_v.at[b], acc.at[dst_v.at[ch]], add=True)

                @pl.when(ch + NBUF < SEG_ROWS)
                def _():
                    _gather(ch + NBUF, b)
            return 0
        lax.fori_loop(0, SEG_ROWS // NBUF, _mb, 0)

    plsc.subcore_barrier()
    pltpu.sync_copy(acc.at[pl.ds(s * STRIPE, STRIPE)],
                    out_hbm.at[pl.ds(c * ACC_ROWS + s * STRIPE, STRIPE)])


_sc_kernel = functools.partial(
    pl.kernel,
    out_type=jax.ShapeDtypeStruct((NC * ACC_ROWS, HALF), jnp.float32),
    mesh=plsc.VectorSubcoreMesh(core_axis_name="c", subcore_axis_name="s",
                                num_cores=NC, num_subcores=NS),
    scratch_types=[
        pltpu.VMEM((SEG_ROWS, CHUNK), jnp.int32),
        pltpu.VMEM((SEG_ROWS, CHUNK), jnp.int32),
        pltpu.VMEM((NBUF, CHUNK, HALF), jnp.float32),
        pltpu.VMEM_SHARED((ACC_ROWS, HALF), jnp.float32),
        pltpu.SemaphoreType.DMA, pltpu.SemaphoreType.DMA,
    ],
)(_sc_kernel_body)


@jax.jit
def kernel(x, edge_index, edge_types, Ws, bs):
    src = edge_index[0].astype(jnp.int32)
    dst = edge_index[1].astype(jnp.int32)
    et = edge_types.astype(jnp.int32)

    pad = E_PAD - N_EDGES
    src2 = jnp.concatenate([src, jnp.zeros((pad,), jnp.int32)]).reshape(-1, CHUNK)
    et2 = jnp.concatenate([et, jnp.zeros((pad,), jnp.int32)]).reshape(-1, CHUNK)
    dst_p = jnp.concatenate([dst, jnp.full((pad,), N_NODES, jnp.int32)]).reshape(-1, CHUNK)

    table, gidx = _make_table_gidx(x, Ws, bs, src2, et2)
    o = _sc_kernel(table, gidx, dst_p)
    return jnp.concatenate([o[:N_NODES], o[ACC_ROWS:ACC_ROWS + N_NODES]], axis=1)


# speedup vs baseline: 1.1332x; 1.0051x over previous
import functools

import jax
import jax.numpy as jnp
from jax import lax
from jax.experimental import pallas as pl
from jax.experimental.pallas import tpu as pltpu
from jax.experimental.pallas import tpu_sc as plsc

N_NODES = 10000
N_EDGES = 160000
D_IN = 256
D_OUT = 256
N_ETYPES = 4

NC = 2
NS = 16
LANES = 16

CHUNK = 128
NROWS = 1280
E_PAD = NROWS * CHUNK
ROWS_PER_TILE = NROWS // NS
NSEG = 5
SEG_ROWS = ROWS_PER_TILE // NSEG
NBUF = 2
ACC_ROWS = 10240
STRIPE = ACC_ROWS // NS
HALF = D_OUT // 2

BN = 10000
EBLK = 320


def _tc_body(x_ref, w_ref, b_ref, src_ref, et_ref, h_ref, g_ref):
    acc = lax.dot_general(
        x_ref[...], w_ref[0],
        dimension_numbers=(((1,), (1,)), ((), ())),
        preferred_element_type=jnp.float32,
    )
    h_ref[0] = acc + b_ref[0, 0, 0][None, :]
    c = pl.program_id(1)
    g_ref[0] = et_ref[...] * N_NODES + src_ref[...] + c * (N_ETYPES * N_NODES)


def _make_table_gidx(x, Ws, bs, src2, et2):
    bs4 = jnp.broadcast_to(
        bs.reshape(N_ETYPES, NC, HALF).transpose(1, 0, 2)[:, :, None, :],
        (NC, N_ETYPES, 8, HALF))
    nb = N_NODES // BN
    erows = NROWS // nb
    grid = (nb, NC, N_ETYPES)
    H2, gidx = pl.pallas_call(
        _tc_body,
        grid=grid,
        in_specs=[
            pl.BlockSpec((BN, D_IN), lambda n, c, t: (n, 0)),
            pl.BlockSpec((1, HALF, D_IN), lambda n, c, t: (t, c, 0)),
            pl.BlockSpec((1, 1, 8, HALF), lambda n, c, t: (c, t, 0, 0)),
            pl.BlockSpec((erows, CHUNK), lambda n, c, t: (n, 0)),
            pl.BlockSpec((erows, CHUNK), lambda n, c, t: (n, 0)),
        ],
        out_specs=[
            pl.BlockSpec((1, BN, HALF), lambda n, c, t: (c, t * nb + n, 0)),
            pl.BlockSpec((1, erows, CHUNK), lambda n, c, t: (c, n, 0)),
        ],
        out_shape=[
            jax.ShapeDtypeStruct((NC, N_ETYPES * N_NODES, HALF), jnp.float32),
            jax.ShapeDtypeStruct((NC, NROWS, CHUNK), jnp.int32),
        ],
    )(x, Ws, bs4, src2, et2)
    return (H2.reshape(NC * N_ETYPES * N_NODES, HALF),
            gidx.reshape(NC * NROWS, CHUNK))


def _sc_kernel_body(h_hbm, gidx_hbm, dst_hbm, out_hbm,
                    gidx_v, dst_v, rows_v, acc, gsem0, gsem1, stsem):
    c = lax.axis_index("c")
    s = lax.axis_index("s")
    base = s * ROWS_PER_TILE
    gsems = [gsem0, gsem1]

    def _gather(p, ch, b):
        pltpu.async_copy(h_hbm.at[gidx_v.at[p, ch]], rows_v.at[b], gsems[b])

    def _gather_wait(p, ch, b):
        pltpu.make_async_copy(h_hbm.at[gidx_v.at[p, ch]], rows_v.at[b],
                              gsems[b]).wait()

    def _stage_start(seg, p):
        r0 = base + seg * SEG_ROWS
        pltpu.async_copy(gidx_hbm.at[pl.ds(c * NROWS + r0, SEG_ROWS)],
                         gidx_v.at[p], stsem)
        pltpu.async_copy(dst_hbm.at[pl.ds(r0, SEG_ROWS)], dst_v.at[p], stsem)

    def _stage_wait(seg, p):
        r0 = base + seg * SEG_ROWS
        pltpu.make_async_copy(gidx_hbm.at[pl.ds(c * NROWS + r0, SEG_ROWS)],
                              gidx_v.at[p], stsem).wait()
        pltpu.make_async_copy(dst_hbm.at[pl.ds(r0, SEG_ROWS)],
                              dst_v.at[p], stsem).wait()

    pltpu.sync_copy(gidx_hbm.at[pl.ds(c * NROWS + base, SEG_ROWS)], gidx_v.at[0])
    pltpu.sync_copy(dst_hbm.at[pl.ds(base, SEG_ROWS)], dst_v.at[0])
    _gather(0, 0, 0)
    _stage_start(1, 1)

    def _zb(i, _):
        rows_v[1, i // 8, pl.ds((i % 8) * LANES, LANES)] = jnp.zeros((LANES,), jnp.float32)
        return 0
    lax.fori_loop(0, CHUNK * 8, _zb, 0)

    def _za(m, _):
        pltpu.sync_copy(rows_v.at[1], acc.at[pl.ds(s * STRIPE + m * CHUNK, CHUNK)])
        return 0
    lax.fori_loop(0, STRIPE // CHUNK, _za, 0)

    plsc.subcore_barrier()
    _gather(0, 1, 1)

    for seg in range(NSEG):
        p = seg % 2

        def _mb(j, _):
            for b in range(NBUF):
                ch = j * NBUF + b
                _gather_wait(p, ch, b)
                pltpu.sync_copy(rows_v.at[b], acc.at[dst_v.at[p, ch]], add=True)
                _gather(p, ch + NBUF, b)
            return 0
        lax.fori_loop(0, SEG_ROWS // NBUF - 1, _mb, 0)

        if seg + 1 < NSEG:
            _stage_wait(seg + 1, 1 - p)
            for b in range(NBUF):
                ch = SEG_ROWS - NBUF + b
                _gather_wait(p, ch, b)
                pltpu.sync_copy(rows_v.at[b], acc.at[dst_v.at[p, ch]], add=True)
                _gather(1 - p, b, b)
            if seg + 2 < NSEG:
                _stage_start(seg + 2, p)
        else:
            for b in range(NBUF):
                ch = SEG_ROWS - NBUF + b
                _gather_wait(p, ch, b)
                pltpu.sync_copy(rows_v.at[b], acc.at[dst_v.at[p, ch]], add=True)

    plsc.subcore_barrier()
    pltpu.sync_copy(acc.at[pl.ds(s * STRIPE, STRIPE)],
                    out_hbm.at[pl.ds(c * ACC_ROWS + s * STRIPE, STRIPE)])


_sc_kernel = functools.partial(
    pl.kernel,
    out_type=jax.ShapeDtypeStruct((NC * ACC_ROWS, HALF), jnp.float32),
    mesh=plsc.VectorSubcoreMesh(core_axis_name="c", subcore_axis_name="s",
                                num_cores=NC, num_subcores=NS),
    scratch_types=[
        pltpu.VMEM((2, SEG_ROWS, CHUNK), jnp.int32),
        pltpu.VMEM((2, SEG_ROWS, CHUNK), jnp.int32),
        pltpu.VMEM((NBUF, CHUNK, HALF), jnp.float32),
        pltpu.VMEM_SHARED((ACC_ROWS, HALF), jnp.float32),
        pltpu.SemaphoreType.DMA, pltpu.SemaphoreType.DMA,
        pltpu.SemaphoreType.DMA,
    ],
)(_sc_kernel_body)


@jax.jit
def kernel(x, edge_index, edge_types, Ws, bs):
    src = edge_index[0].astype(jnp.int32)
    dst = edge_index[1].astype(jnp.int32)
    et = edge_types.astype(jnp.int32)

    pad = E_PAD - N_EDGES
    src2 = jnp.concatenate([src, jnp.zeros((pad,), jnp.int32)]).reshape(-1, CHUNK)
    et2 = jnp.concatenate([et, jnp.zeros((pad,), jnp.int32)]).reshape(-1, CHUNK)
    dst_p = jnp.concatenate([dst, jnp.full((pad,), N_NODES, jnp.int32)]).reshape(-1, CHUNK)

    table, gidx = _make_table_gidx(x, Ws, bs, src2, et2)
    o = _sc_kernel(table, gidx, dst_p)
    return jnp.concatenate([o[:N_NODES], o[ACC_ROWS:ACC_ROWS + N_NODES]], axis=1)


# speedup vs baseline: 1.1334x; 1.0002x over previous
import functools

import jax
import jax.numpy as jnp
from jax import lax
from jax.experimental import pallas as pl
from jax.experimental.pallas import tpu as pltpu
from jax.experimental.pallas import tpu_sc as plsc

N_NODES = 10000
N_EDGES = 160000
D_IN = 256
D_OUT = 256
N_ETYPES = 4

NC = 2
NS = 16
LANES = 16

CHUNK = 128
NROWS = 1280
E_PAD = NROWS * CHUNK
ROWS_PER_TILE = NROWS // NS
NSEG = 5
SEG_ROWS = ROWS_PER_TILE // NSEG
NBUF = 2
ACC_ROWS = 10240
STRIPE = ACC_ROWS // NS
HALF = D_OUT // 2

BN = 10000
EBLK = 320


def _tc_body(x_ref, w_ref, b_ref, src_ref, et_ref, h_ref, g_ref):
    acc = lax.dot_general(
        x_ref[...], w_ref[0],
        dimension_numbers=(((1,), (1,)), ((), ())),
        preferred_element_type=jnp.float32,
    )
    h_ref[0] = acc + b_ref[0, 0, 0][None, :]
    c = pl.program_id(1)
    g_ref[0] = et_ref[...] * N_NODES + src_ref[...] + c * (N_ETYPES * N_NODES)


def _make_table_gidx(x, Ws, bs, src2, et2):
    bs4 = jnp.broadcast_to(
        bs.reshape(N_ETYPES, NC, HALF).transpose(1, 0, 2)[:, :, None, :],
        (NC, N_ETYPES, 8, HALF))
    nb = N_NODES // BN
    erows = NROWS // nb
    grid = (nb, NC, N_ETYPES)
    H2, gidx = pl.pallas_call(
        _tc_body,
        grid=grid,
        in_specs=[
            pl.BlockSpec((BN, D_IN), lambda n, c, t: (n, 0)),
            pl.BlockSpec((1, HALF, D_IN), lambda n, c, t: (t, c, 0)),
            pl.BlockSpec((1, 1, 8, HALF), lambda n, c, t: (c, t, 0, 0)),
            pl.BlockSpec((erows, CHUNK), lambda n, c, t: (n, 0)),
            pl.BlockSpec((erows, CHUNK), lambda n, c, t: (n, 0)),
        ],
        out_specs=[
            pl.BlockSpec((1, BN, HALF), lambda n, c, t: (c, t * nb + n, 0)),
            pl.BlockSpec((1, erows, CHUNK), lambda n, c, t: (c, n, 0)),
        ],
        out_shape=[
            jax.ShapeDtypeStruct((NC, N_ETYPES * N_NODES, HALF), jnp.float32),
            jax.ShapeDtypeStruct((NC, NROWS, CHUNK), jnp.int32),
        ],
    )(x, Ws, bs4, src2, et2)
    return (H2.reshape(NC * N_ETYPES * N_NODES, HALF),
            gidx.reshape(NC * NROWS, CHUNK))


def _sc_kernel_body(h_hbm, gidx_hbm, dst_hbm, out_hbm,
                    gidx_v, dst_v, rows_v, acc, gsem0, gsem1, stsem):
    c = lax.axis_index("c")
    s = lax.axis_index("s")
    base = s * ROWS_PER_TILE
    gsems = [gsem0, gsem1]

    def _gather(p, ch, b):
        pltpu.async_copy(h_hbm.at[gidx_v.at[p, ch]], rows_v.at[b], gsems[b],
                         priority=b)

    def _gather_wait(p, ch, b):
        pltpu.make_async_copy(h_hbm.at[gidx_v.at[p, ch]], rows_v.at[b],
                              gsems[b]).wait()

    def _stage_start(seg, p):
        r0 = base + seg * SEG_ROWS
        pltpu.async_copy(gidx_hbm.at[pl.ds(c * NROWS + r0, SEG_ROWS)],
                         gidx_v.at[p], stsem)
        pltpu.async_copy(dst_hbm.at[pl.ds(r0, SEG_ROWS)], dst_v.at[p], stsem)

    def _stage_wait(seg, p):
        r0 = base + seg * SEG_ROWS
        pltpu.make_async_copy(gidx_hbm.at[pl.ds(c * NROWS + r0, SEG_ROWS)],
                              gidx_v.at[p], stsem).wait()
        pltpu.make_async_copy(dst_hbm.at[pl.ds(r0, SEG_ROWS)],
                              dst_v.at[p], stsem).wait()

    pltpu.sync_copy(gidx_hbm.at[pl.ds(c * NROWS + base, SEG_ROWS)], gidx_v.at[0])
    pltpu.sync_copy(dst_hbm.at[pl.ds(base, SEG_ROWS)], dst_v.at[0])
    _gather(0, 0, 0)
    _stage_start(1, 1)

    def _zb(i, _):
        rows_v[1, i // 8, pl.ds((i % 8) * LANES, LANES)] = jnp.zeros((LANES,), jnp.float32)
        return 0
    lax.fori_loop(0, CHUNK * 8, _zb, 0)

    def _za(m, _):
        pltpu.sync_copy(rows_v.at[1], acc.at[pl.ds(s * STRIPE + m * CHUNK, CHUNK)])
        return 0
    lax.fori_loop(0, STRIPE // CHUNK, _za, 0)

    plsc.subcore_barrier()
    _gather(0, 1, 1)

    for seg in range(NSEG):
        p = seg % 2

        def _mb(j, _):
            for b in range(NBUF):
                ch = j * NBUF + b
                _gather_wait(p, ch, b)
                pltpu.sync_copy(rows_v.at[b], acc.at[dst_v.at[p, ch]], add=True)
                _gather(p, ch + NBUF, b)
            return 0
        lax.fori_loop(0, SEG_ROWS // NBUF - 1, _mb, 0)

        if seg + 1 < NSEG:
            _stage_wait(seg + 1, 1 - p)
            for b in range(NBUF):
                ch = SEG_ROWS - NBUF + b
                _gather_wait(p, ch, b)
                pltpu.sync_copy(rows_v.at[b], acc.at[dst_v.at[p, ch]], add=True)
                _gather(1 - p, b, b)
            if seg + 2 < NSEG:
                _stage_start(seg + 2, p)
        else:
            for b in range(NBUF):
                ch = SEG_ROWS - NBUF + b
                _gather_wait(p, ch, b)
                pltpu.sync_copy(rows_v.at[b], acc.at[dst_v.at[p, ch]], add=True)

    plsc.subcore_barrier()
    pltpu.sync_copy(acc.at[pl.ds(s * STRIPE, STRIPE)],
                    out_hbm.at[pl.ds(c * ACC_ROWS + s * STRIPE, STRIPE)])


_sc_kernel = functools.partial(
    pl.kernel,
    out_type=jax.ShapeDtypeStruct((NC * ACC_ROWS, HALF), jnp.float32),
    mesh=plsc.VectorSubcoreMesh(core_axis_name="c", subcore_axis_name="s",
                                num_cores=NC, num_subcores=NS),
    scratch_types=[
        pltpu.VMEM((2, SEG_ROWS, CHUNK), jnp.int32),
        pltpu.VMEM((2, SEG_ROWS, CHUNK), jnp.int32),
        pltpu.VMEM((NBUF, CHUNK, HALF), jnp.float32),
        pltpu.VMEM_SHARED((ACC_ROWS, HALF), jnp.float32),
        pltpu.SemaphoreType.DMA, pltpu.SemaphoreType.DMA,
        pltpu.SemaphoreType.DMA,
    ],
)(_sc_kernel_body)


@jax.jit
def kernel(x, edge_index, edge_types, Ws, bs):
    src = edge_index[0].astype(jnp.int32)
    dst = edge_index[1].astype(jnp.int32)
    et = edge_types.astype(jnp.int32)

    pad = E_PAD - N_EDGES
    src2 = jnp.concatenate([src, jnp.zeros((pad,), jnp.int32)]).reshape(-1, CHUNK)
    et2 = jnp.concatenate([et, jnp.zeros((pad,), jnp.int32)]).reshape(-1, CHUNK)
    dst_p = jnp.concatenate([dst, jnp.full((pad,), N_NODES, jnp.int32)]).reshape(-1, CHUNK)

    table, gidx = _make_table_gidx(x, Ws, bs, src2, et2)
    o = _sc_kernel(table, gidx, dst_p)
    return jnp.concatenate([o[:N_NODES], o[ACC_ROWS:ACC_ROWS + N_NODES]], axis=1)
